# baseline (device time: 134774 ns/iter reference)
import jax
import jax.numpy as jnp
from jax import lax
from jax.experimental import pallas as pl
from jax.experimental.pallas import tpu as pltpu

N_CHUNKS = 8
N_SFX_CHUNKS = 8


def kernel(x, W):
    t, d = x.shape
    _, v_loc = W.shape
    v_glob = 2 * v_loc
    gchunk = v_loc // N_CHUNKS
    schunk = v_glob // N_SFX_CHUNKS

    def body(x_ref, w_hbm, out_ref, w_buf, send_buf, recv_buf,
             w_sems, send_sems, recv_sems):
        my_x = lax.axis_index("x")
        my_y = lax.axis_index("y")
        nbr = (my_x, 1 - my_y)
        col0 = my_y * v_loc
        rcol0 = (1 - my_y) * v_loc

        barrier_sem = pltpu.get_barrier_semaphore()
        pl.semaphore_signal(
            barrier_sem, inc=1, device_id=nbr,
            device_id_type=pl.DeviceIdType.MESH,
        )
        pl.semaphore_wait(barrier_sem, 1)

        def w_copy(c, slot):
            return pltpu.make_async_copy(
                w_hbm.at[:, pl.ds(c * gchunk, gchunk)],
                w_buf.at[slot],
                w_sems.at[slot],
            )

        def rdma(c):
            sl = pl.ds(c * gchunk, gchunk)
            return pltpu.make_async_remote_copy(
                src_ref=send_buf.at[:, sl],
                dst_ref=recv_buf.at[:, sl],
                send_sem=send_sems.at[c],
                recv_sem=recv_sems.at[c],
                device_id=nbr,
                device_id_type=pl.DeviceIdType.MESH,
            )

        xv = x_ref[:, :]
        s = jnp.zeros((t, 1), dtype=jnp.float32)
        w_copy(0, 0).start()
        for c in range(N_CHUNKS):
            slot = c % 2
            if c + 1 < N_CHUNKS:
                w_copy(c + 1, (c + 1) % 2).start()
            w_copy(c, slot).wait()
            z = jnp.dot(xv, w_buf[slot], preferred_element_type=jnp.float32)
            p = jnp.exp(z)
            sl = pl.ds(c * gchunk, gchunk)
            out_ref[:, pl.ds(col0 + c * gchunk, gchunk)] = p
            send_buf[:, sl] = p.astype(jnp.bfloat16)
            rdma(c).start()
            s = s + jnp.sum(p, axis=1, keepdims=True)

        for c in range(N_CHUNKS):
            rdma(c).wait_recv()
            pc = recv_buf[:, pl.ds(c * gchunk, gchunk)].astype(jnp.float32)
            out_ref[:, pl.ds(rcol0 + c * gchunk, gchunk)] = pc
            s = s + jnp.sum(pc, axis=1, keepdims=True)

        for c in range(N_CHUNKS):
            rdma(c).wait_send()

        r = 1.0 / s
        for c in range(N_SFX_CHUNKS):
            sl = pl.ds(c * schunk, schunk)
            out_ref[:, sl] = out_ref[:, sl] * r

    return pl.pallas_call(
        body,
        out_shape=jax.ShapeDtypeStruct((t, v_glob), jnp.float32),
        in_specs=[
            pl.BlockSpec(memory_space=pltpu.VMEM),
            pl.BlockSpec(memory_space=pltpu.MemorySpace.HBM),
        ],
        out_specs=pl.BlockSpec(memory_space=pltpu.VMEM),
        scratch_shapes=[
            pltpu.VMEM((2, d, gchunk), jnp.float32),
            pltpu.VMEM((t, v_loc), jnp.bfloat16),
            pltpu.VMEM((t, v_loc), jnp.bfloat16),
            pltpu.SemaphoreType.DMA((2,)),
            pltpu.SemaphoreType.DMA((N_CHUNKS,)),
            pltpu.SemaphoreType.DMA((N_CHUNKS,)),
        ],
        compiler_params=pltpu.CompilerParams(
            collective_id=0,
            vmem_limit_bytes=62 * 1024 * 1024,
        ),
    )(x, W)


# device time: 133455 ns/iter; 1.0099x vs baseline; 1.0099x over previous
import jax
import jax.numpy as jnp
from jax import lax
from jax.experimental import pallas as pl
from jax.experimental.pallas import tpu as pltpu

N_CHUNKS = 8
N_OUT_CHUNKS = 16


def kernel(x, W):
    t, d = x.shape
    _, v_loc = W.shape
    v_glob = 2 * v_loc
    gchunk = v_loc // N_CHUNKS
    ochunk = v_glob // N_OUT_CHUNKS

    def body(x_ref, w_hbm, out_hbm, acc, w_buf, send_buf, recv_buf,
             w_sems, send_sems, recv_sems, out_sems):
        my_x = lax.axis_index("x")
        my_y = lax.axis_index("y")
        nbr = (my_x, 1 - my_y)
        col0 = my_y * v_loc
        rcol0 = (1 - my_y) * v_loc

        barrier_sem = pltpu.get_barrier_semaphore()
        pl.semaphore_signal(
            barrier_sem, inc=1, device_id=nbr,
            device_id_type=pl.DeviceIdType.MESH,
        )
        pl.semaphore_wait(barrier_sem, 1)

        def w_copy(c, slot):
            return pltpu.make_async_copy(
                w_hbm.at[:, pl.ds(c * gchunk, gchunk)],
                w_buf.at[slot],
                w_sems.at[slot],
            )

        def rdma(c):
            sl = pl.ds(c * gchunk, gchunk)
            return pltpu.make_async_remote_copy(
                src_ref=send_buf.at[:, sl],
                dst_ref=recv_buf.at[:, sl],
                send_sem=send_sems.at[c],
                recv_sem=recv_sems.at[c],
                device_id=nbr,
                device_id_type=pl.DeviceIdType.MESH,
            )

        def out_copy(c):
            sl = pl.ds(c * ochunk, ochunk)
            return pltpu.make_async_copy(
                acc.at[:, sl], out_hbm.at[:, sl], out_sems.at[c],
            )

        xv = x_ref[:, :]
        s = jnp.zeros((t, 1), dtype=jnp.float32)
        w_copy(0, 0).start()
        for c in range(N_CHUNKS):
            slot = c % 2
            if c + 1 < N_CHUNKS:
                w_copy(c + 1, (c + 1) % 2).start()
            w_copy(c, slot).wait()
            z = jnp.dot(xv, w_buf[slot], preferred_element_type=jnp.float32)
            p = jnp.exp(z)
            sl = pl.ds(c * gchunk, gchunk)
            acc[:, pl.ds(col0 + c * gchunk, gchunk)] = p
            send_buf[:, sl] = p.astype(jnp.bfloat16)
            rdma(c).start()
            s = s + jnp.sum(p, axis=1, keepdims=True)

        for c in range(N_CHUNKS):
            rdma(c).wait_recv()
            pc = recv_buf[:, pl.ds(c * gchunk, gchunk)].astype(jnp.float32)
            acc[:, pl.ds(rcol0 + c * gchunk, gchunk)] = pc
            s = s + jnp.sum(pc, axis=1, keepdims=True)

        for c in range(N_CHUNKS):
            rdma(c).wait_send()

        r = 1.0 / s
        for c in range(N_OUT_CHUNKS):
            sl = pl.ds(c * ochunk, ochunk)
            acc[:, sl] = acc[:, sl] * r
            out_copy(c).start()
        for c in range(N_OUT_CHUNKS):
            out_copy(c).wait()

    return pl.pallas_call(
        body,
        out_shape=jax.ShapeDtypeStruct((t, v_glob), jnp.float32),
        in_specs=[
            pl.BlockSpec(memory_space=pltpu.VMEM),
            pl.BlockSpec(memory_space=pltpu.MemorySpace.HBM),
        ],
        out_specs=pl.BlockSpec(memory_space=pltpu.MemorySpace.HBM),
        scratch_shapes=[
            pltpu.VMEM((t, v_glob), jnp.float32),
            pltpu.VMEM((2, d, gchunk), jnp.float32),
            pltpu.VMEM((t, v_loc), jnp.bfloat16),
            pltpu.VMEM((t, v_loc), jnp.bfloat16),
            pltpu.SemaphoreType.DMA((2,)),
            pltpu.SemaphoreType.DMA((N_CHUNKS,)),
            pltpu.SemaphoreType.DMA((N_CHUNKS,)),
            pltpu.SemaphoreType.DMA((N_OUT_CHUNKS,)),
        ],
        compiler_params=pltpu.CompilerParams(
            collective_id=0,
            vmem_limit_bytes=62 * 1024 * 1024,
        ),
    )(x, W)


# device time: 102602 ns/iter; 1.3136x vs baseline; 1.3007x over previous
import jax
import jax.numpy as jnp
from jax import lax
from jax.experimental import pallas as pl
from jax.experimental.pallas import tpu as pltpu

N_CHUNKS = 8
N_OUT_CHUNKS = 16


def kernel(x, W):
    t, d = x.shape
    _, v_loc = W.shape
    v_glob = 2 * v_loc
    rhalf = t // 2
    gchunk = v_loc // N_CHUNKS
    ochunk = v_glob // N_OUT_CHUNKS

    def body(x_ref, w_hbm, out_hbm, acc, w_buf, send_buf, recv_buf, sums,
             w_sems, ysend_sems, yrecv_sems, xsend_sems, xrecv_sems,
             out_sems):
        my_x = lax.axis_index("x")
        my_y = lax.axis_index("y")
        ynbr = (my_x, 1 - my_y)
        xnbr = (1 - my_x, my_y)
        col0 = my_y * v_loc
        rcol0 = (1 - my_y) * v_loc
        myrow = pl.ds(my_x * rhalf, rhalf)
        otrow = pl.ds((1 - my_x) * rhalf, rhalf)

        barrier_sem = pltpu.get_barrier_semaphore()
        for nbr in (ynbr, xnbr):
            pl.semaphore_signal(
                barrier_sem, inc=1, device_id=nbr,
                device_id_type=pl.DeviceIdType.MESH,
            )
        pl.semaphore_wait(barrier_sem, 2)

        def w_copy(c, slot):
            return pltpu.make_async_copy(
                w_hbm.at[:, pl.ds(c * gchunk, gchunk)],
                w_buf.at[slot],
                w_sems.at[slot],
            )

        def rdma_y(c):
            sl = (myrow, pl.ds(c * gchunk, gchunk))
            return pltpu.make_async_remote_copy(
                src_ref=send_buf.at[sl],
                dst_ref=recv_buf.at[sl],
                send_sem=ysend_sems.at[c],
                recv_sem=yrecv_sems.at[c],
                device_id=ynbr,
                device_id_type=pl.DeviceIdType.MESH,
            )

        def rdma_x(c):
            sl = (myrow, pl.ds(c * gchunk, gchunk))
            return pltpu.make_async_remote_copy(
                src_ref=recv_buf.at[sl],
                dst_ref=recv_buf.at[sl],
                send_sem=xsend_sems.at[c],
                recv_sem=xrecv_sems.at[c],
                device_id=xnbr,
                device_id_type=pl.DeviceIdType.MESH,
            )

        def out_copy(c):
            sl = pl.ds(c * ochunk, ochunk)
            return pltpu.make_async_copy(
                acc.at[:, sl], out_hbm.at[:, sl], out_sems.at[c],
            )

        xv = x_ref[:, :]
        s_loc = jnp.zeros((t, 1), dtype=jnp.float32)
        w_copy(0, 0).start()
        for c in range(N_CHUNKS):
            slot = c % 2
            if c + 1 < N_CHUNKS:
                w_copy(c + 1, (c + 1) % 2).start()
            w_copy(c, slot).wait()
            z = jnp.dot(xv, w_buf[slot], preferred_element_type=jnp.float32)
            p = jnp.exp(z)
            sl = pl.ds(c * gchunk, gchunk)
            acc[:, pl.ds(col0 + c * gchunk, gchunk)] = p
            send_buf[:, sl] = p.astype(jnp.bfloat16)
            rdma_y(c).start()
            s_loc = s_loc + jnp.sum(p, axis=1, keepdims=True)

        s_y = jnp.zeros((rhalf, 1), dtype=jnp.float32)
        for c in range(N_CHUNKS):
            rdma_y(c).wait_recv()
            rdma_x(c).start()
            pc = recv_buf[myrow, pl.ds(c * gchunk, gchunk)].astype(jnp.float32)
            acc[myrow, pl.ds(rcol0 + c * gchunk, gchunk)] = pc
            s_y = s_y + jnp.sum(pc, axis=1, keepdims=True)

        s_x = jnp.zeros((rhalf, 1), dtype=jnp.float32)
        for c in range(N_CHUNKS):
            rdma_x(c).wait_recv()
            qc = recv_buf[otrow, pl.ds(c * gchunk, gchunk)].astype(jnp.float32)
            acc[otrow, pl.ds(rcol0 + c * gchunk, gchunk)] = qc
            s_x = s_x + jnp.sum(qc, axis=1, keepdims=True)

        for c in range(N_CHUNKS):
            rdma_y(c).wait_send()
            rdma_x(c).wait_send()

        sums[myrow, :] = s_y
        sums[otrow, :] = s_x
        r = 1.0 / (s_loc + sums[:, :])
        for c in range(N_OUT_CHUNKS):
            sl = pl.ds(c * ochunk, ochunk)
            acc[:, sl] = acc[:, sl] * r
            out_copy(c).start()
        for c in range(N_OUT_CHUNKS):
            out_copy(c).wait()

    return pl.pallas_call(
        body,
        out_shape=jax.ShapeDtypeStruct((t, v_glob), jnp.float32),
        in_specs=[
            pl.BlockSpec(memory_space=pltpu.VMEM),
            pl.BlockSpec(memory_space=pltpu.MemorySpace.HBM),
        ],
        out_specs=pl.BlockSpec(memory_space=pltpu.MemorySpace.HBM),
        scratch_shapes=[
            pltpu.VMEM((t, v_glob), jnp.float32),
            pltpu.VMEM((2, d, gchunk), jnp.float32),
            pltpu.VMEM((t, v_loc), jnp.bfloat16),
            pltpu.VMEM((t, v_loc), jnp.bfloat16),
            pltpu.VMEM((t, 1), jnp.float32),
            pltpu.SemaphoreType.DMA((2,)),
            pltpu.SemaphoreType.DMA((N_CHUNKS,)),
            pltpu.SemaphoreType.DMA((N_CHUNKS,)),
            pltpu.SemaphoreType.DMA((N_CHUNKS,)),
            pltpu.SemaphoreType.DMA((N_CHUNKS,)),
            pltpu.SemaphoreType.DMA((N_OUT_CHUNKS,)),
        ],
        compiler_params=pltpu.CompilerParams(
            collective_id=0,
            vmem_limit_bytes=62 * 1024 * 1024,
        ),
    )(x, W)
